# baseline (speedup 1.0000x reference)
import jax
import jax.numpy as jnp
from jax.experimental import pallas as pl
from jax.experimental.pallas import tpu as pltpu

_B = 16384
_D = 128
_BLK = 512
_GRID = _B // _BLK


def _fused_kernel(u_full_ref, d2_full_ref, d1_ref, d2_ref, v1_ref, v2_ref,
                  out_ref):
    u = u_full_ref[...]
    m1 = jnp.max(u)
    m2 = jnp.max(u - d2_full_ref[...])
    out_ref[...] = (v2_ref[...] * jnp.minimum(d2_ref[...], m1)
                    + v1_ref[...] * jnp.minimum(d1_ref[...], m2))


def kernel(u, d1, d2, v1, v2):
    u_r = u.reshape(128, 128)
    d2_r = d2.reshape(128, 128)
    return pl.pallas_call(
        _fused_kernel,
        grid=(_GRID,),
        in_specs=[
            pl.BlockSpec((128, 128), lambda i: (0, 0)),
            pl.BlockSpec((128, 128), lambda i: (0, 0)),
            pl.BlockSpec((_BLK, 1), lambda i: (i, 0)),
            pl.BlockSpec((_BLK, 1), lambda i: (i, 0)),
            pl.BlockSpec((_BLK, _D), lambda i: (i, 0)),
            pl.BlockSpec((_BLK, _D), lambda i: (i, 0)),
        ],
        out_specs=pl.BlockSpec((_BLK, _D), lambda i: (i, 0)),
        out_shape=jax.ShapeDtypeStruct((_B, _D), jnp.float32),
    )(u_r, d2_r, d1, d2, v1, v2)


# speedup vs baseline: 3.0957x; 3.0957x over previous
import jax
import jax.numpy as jnp
from jax.experimental import pallas as pl
from jax.experimental.pallas import tpu as pltpu

_B = 16384
_D = 128
_R = _B // 128
_BS = 16
_GRID = _R // _BS


def _fused_kernel(u_full_ref, d2_full_ref, d1_ref, d2_ref, v1_ref, v2_ref,
                  out_ref, m_ref):
    @pl.when(pl.program_id(0) == 0)
    def _():
        u = u_full_ref[...]
        m_ref[0] = jnp.max(u)
        m_ref[1] = jnp.max(u - d2_full_ref[...])

    w2 = jnp.minimum(d2_ref[...], m_ref[0])
    w1 = jnp.minimum(d1_ref[...], m_ref[1])
    out_ref[...] = (v2_ref[...] * w2[:, :, None]
                    + v1_ref[...] * w1[:, :, None])


def kernel(u, d1, d2, v1, v2):
    u_r = u.reshape(128, 128)
    d1_r = d1.reshape(128, 128)
    d2_r = d2.reshape(128, 128)
    v1_r = v1.reshape(128, 128, _D)
    v2_r = v2.reshape(128, 128, _D)
    out = pl.pallas_call(
        _fused_kernel,
        grid=(_GRID,),
        in_specs=[
            pl.BlockSpec((128, 128), lambda i: (0, 0)),
            pl.BlockSpec((128, 128), lambda i: (0, 0)),
            pl.BlockSpec((_BS, 128), lambda i: (i, 0)),
            pl.BlockSpec((_BS, 128), lambda i: (i, 0)),
            pl.BlockSpec((_BS, 128, _D), lambda i: (i, 0, 0)),
            pl.BlockSpec((_BS, 128, _D), lambda i: (i, 0, 0)),
        ],
        out_specs=pl.BlockSpec((_BS, 128, _D), lambda i: (i, 0, 0)),
        out_shape=jax.ShapeDtypeStruct((128, 128, _D), jnp.float32),
        scratch_shapes=[pltpu.SMEM((2,), jnp.float32)],
    )(u_r, d2_r, d1_r, d2_r, v1_r, v2_r)
    return out.reshape(_B, _D)


# speedup vs baseline: 3.4282x; 1.1074x over previous
import jax
import jax.numpy as jnp
from jax.experimental import pallas as pl
from jax.experimental.pallas import tpu as pltpu

_B = 16384
_D = 128
_R = _B // 128
_BS = 32
_GRID = _R // _BS


def _fused_kernel(u_full_ref, d2_full_ref, d1_ref, d2_ref, v1_ref, v2_ref,
                  out_ref, m_ref):
    @pl.when(pl.program_id(0) == 0)
    def _():
        u = u_full_ref[...]
        m_ref[0] = jnp.max(u)
        m_ref[1] = jnp.max(u - d2_full_ref[...])

    w2 = jnp.minimum(d2_ref[...], m_ref[0])
    w1 = jnp.minimum(d1_ref[...], m_ref[1])
    out_ref[...] = (v2_ref[...] * w2[:, :, None]
                    + v1_ref[...] * w1[:, :, None])


def kernel(u, d1, d2, v1, v2):
    u_r = u.reshape(128, 128)
    d1_r = d1.reshape(128, 128)
    d2_r = d2.reshape(128, 128)
    v1_r = v1.reshape(128, 128, _D)
    v2_r = v2.reshape(128, 128, _D)
    out = pl.pallas_call(
        _fused_kernel,
        grid=(_GRID,),
        in_specs=[
            pl.BlockSpec((128, 128), lambda i: (0, 0)),
            pl.BlockSpec((128, 128), lambda i: (0, 0)),
            pl.BlockSpec((_BS, 128), lambda i: (i, 0)),
            pl.BlockSpec((_BS, 128), lambda i: (i, 0)),
            pl.BlockSpec((_BS, 128, _D), lambda i: (i, 0, 0)),
            pl.BlockSpec((_BS, 128, _D), lambda i: (i, 0, 0)),
        ],
        out_specs=pl.BlockSpec((_BS, 128, _D), lambda i: (i, 0, 0)),
        out_shape=jax.ShapeDtypeStruct((128, 128, _D), jnp.float32),
        scratch_shapes=[pltpu.SMEM((2,), jnp.float32)],
    )(u_r, d2_r, d1_r, d2_r, v1_r, v2_r)
    return out.reshape(_B, _D)


# speedup vs baseline: 3.5661x; 1.0402x over previous
import jax
import jax.numpy as jnp
from jax.experimental import pallas as pl
from jax.experimental.pallas import tpu as pltpu

_B = 16384
_D = 128
_R = _B // 128
_BS = 64
_GRID = _R // _BS


def _fused_kernel(u_full_ref, d2_full_ref, d1_ref, d2_ref, v1_ref, v2_ref,
                  out_ref, m_ref):
    @pl.when(pl.program_id(0) == 0)
    def _():
        u = u_full_ref[...]
        m_ref[0] = jnp.max(u)
        m_ref[1] = jnp.max(u - d2_full_ref[...])

    w2 = jnp.minimum(d2_ref[...], m_ref[0])
    w1 = jnp.minimum(d1_ref[...], m_ref[1])
    out_ref[...] = (v2_ref[...] * w2[:, :, None]
                    + v1_ref[...] * w1[:, :, None])


def kernel(u, d1, d2, v1, v2):
    u_r = u.reshape(128, 128)
    d1_r = d1.reshape(128, 128)
    d2_r = d2.reshape(128, 128)
    v1_r = v1.reshape(128, 128, _D)
    v2_r = v2.reshape(128, 128, _D)
    out = pl.pallas_call(
        _fused_kernel,
        grid=(_GRID,),
        in_specs=[
            pl.BlockSpec((128, 128), lambda i: (0, 0)),
            pl.BlockSpec((128, 128), lambda i: (0, 0)),
            pl.BlockSpec((_BS, 128), lambda i: (i, 0)),
            pl.BlockSpec((_BS, 128), lambda i: (i, 0)),
            pl.BlockSpec((_BS, 128, _D), lambda i: (i, 0, 0)),
            pl.BlockSpec((_BS, 128, _D), lambda i: (i, 0, 0)),
        ],
        out_specs=pl.BlockSpec((_BS, 128, _D), lambda i: (i, 0, 0)),
        out_shape=jax.ShapeDtypeStruct((128, 128, _D), jnp.float32),
        scratch_shapes=[pltpu.SMEM((2,), jnp.float32)],
    )(u_r, d2_r, d1_r, d2_r, v1_r, v2_r)
    return out.reshape(_B, _D)
